# baseline (speedup 1.0000x reference)
import functools
import jax
import jax.numpy as jnp
from jax import lax
from jax.experimental import pallas as pl
from jax.experimental.pallas import tpu as pltpu
from jax.experimental.pallas import tpu_sc as plsc

_BS = 64
_T = 2048
_D = 256
_NC = 2
_NS = 16
_NW = _NC * _NS
_BPW = _BS // _NW
_CHUNK = 256
_NCHUNK = _T // _CHUNK


def _sc_body(table_hbm, out_hbm, chunk_v):
    wid = lax.axis_index("s") * _NC + lax.axis_index("c")
    base = wid * _BPW

    def chunk_loop(ci, carry):
        r0 = ci * _CHUNK
        pltpu.sync_copy(table_hbm.at[pl.ds(r0, _CHUNK)], chunk_v)
        for j in range(_BPW):
            pltpu.sync_copy(chunk_v, out_hbm.at[base + j, pl.ds(r0, _CHUNK)])
        return carry

    lax.fori_loop(0, _NCHUNK, chunk_loop, 0)


def kernel(mask, embed_weight):
    bs, t = mask.shape
    n_embed, d = embed_weight.shape

    mesh = plsc.VectorSubcoreMesh(core_axis_name="c", subcore_axis_name="s")
    k = functools.partial(
        pl.kernel,
        mesh=mesh,
        out_type=jax.ShapeDtypeStruct((bs, t, d), embed_weight.dtype),
        scratch_types=[pltpu.VMEM((_CHUNK, d), embed_weight.dtype)],
    )(_sc_body)
    return k(embed_weight[:t])


# speedup vs baseline: 2.8414x; 2.8414x over previous
import jax
import jax.numpy as jnp
from jax.experimental import pallas as pl
from jax.experimental.pallas import tpu as pltpu

_REP = 2


def _make_body(bs, rep):
    def body(emb_ref, out_ref, stage_ref, copy_sem, out_sem):
        stages = [
            pltpu.make_async_copy(emb_ref, stage_ref.at[r], copy_sem)
            for r in range(rep)
        ]
        for c in stages:
            c.start()
        for c in stages:
            c.wait()
        n = bs // rep
        copies = [
            pltpu.make_async_copy(
                stage_ref, out_ref.at[pl.ds(g * rep, rep)], out_sem
            )
            for g in range(n)
        ]
        for c in copies:
            c.start()
        for c in copies:
            c.wait()

    return body


def kernel(mask, embed_weight):
    bs, t = mask.shape
    n_embed, d = embed_weight.shape
    rep = _REP if bs % _REP == 0 else 1

    out = pl.pallas_call(
        _make_body(bs, rep),
        in_specs=[pl.BlockSpec(memory_space=pltpu.MemorySpace.VMEM)],
        out_specs=pl.BlockSpec(memory_space=pl.ANY),
        out_shape=jax.ShapeDtypeStruct((bs, t, d), embed_weight.dtype),
        scratch_shapes=[
            pltpu.VMEM((rep, t, d), embed_weight.dtype),
            pltpu.SemaphoreType.DMA,
            pltpu.SemaphoreType.DMA,
        ],
    )(embed_weight[:t])
    return out
